# baseline (speedup 1.0000x reference)
import dataclasses
import functools

import jax
import jax.numpy as jnp
from jax import lax
from jax.experimental import pallas as pl
from jax.experimental.pallas import tpu as pltpu
from jax.experimental.pallas import tpu_sc as plsc

_K_INDEX = 15938354

_NB = 2048
_L = 16
_NW = 32
_ELEMS = 4096 * 4096
_PER_W = _ELEMS // _NW
_CHUNK = 32768
_NCHUNK = _PER_W // _CHUNK


def _sc_hist_body(lo_hbm, w_hbm, out_hbm, lo_v, buf0, buf1, hist, sem0, sem1):
    wid = lax.axis_index("s") * 2 + lax.axis_index("c")
    base = wid * _PER_W

    pltpu.sync_copy(lo_hbm, lo_v)
    lovec = lo_v[pl.ds(0, _L)]
    shvec = lo_v[pl.ds(_L, _L)]
    lane = lax.iota(jnp.int32, _L)
    four = jnp.full((_L,), 4, jnp.int32)
    ones = jnp.full((_L,), 1.0, jnp.float32)
    nb = jnp.full((_L,), _NB, jnp.int32)
    zero = jnp.zeros((_L,), jnp.int32)

    def _z(i, carry):
        hist[pl.ds(i * _L, _L)] = jnp.zeros((_L,), jnp.float32)
        return carry
    lax.fori_loop(0, (_NB * _L) // _L, _z, 0, unroll=8)

    bufs = (buf0, buf1)
    sems = (sem0, sem1)
    copies = [None, None]
    copies[0] = pltpu.async_copy(w_hbm.at[pl.ds(base, _CHUNK)], buf0, sem0)

    def _process(buf):
        def _b(i, carry):
            v = buf[pl.ds(i * _L, _L)]
            bits = plsc.bitcast(jnp.abs(v), jnp.int32)
            rel = lax.shift_right_arithmetic(bits - lovec, shvec)
            valid = (rel >= zero) & (rel < nb)
            addr = lax.shift_left(rel, four) + lane
            plsc.addupdate_scatter(hist, [addr], ones, mask=valid)
            return carry
        lax.fori_loop(0, _CHUNK // _L, _b, 0, unroll=8)

    for c in range(_NCHUNK):
        if c + 1 < _NCHUNK:
            copies[(c + 1) % 2] = pltpu.async_copy(
                w_hbm.at[pl.ds(base + (c + 1) * _CHUNK, _CHUNK)],
                bufs[(c + 1) % 2], sems[(c + 1) % 2])
        copies[c % 2].wait()
        _process(bufs[c % 2])

    def _r(i, carry):
        kbase = i * _L
        acc = jnp.zeros((_L,), jnp.float32)
        gidx = lax.shift_left(kbase + lane, four)
        for l in range(_L):
            acc = acc + plsc.load_gather(hist, [gidx + l])
        buf0[pl.ds(kbase, _L)] = acc
        return carry
    lax.fori_loop(0, _NB // _L, _r, 0)

    pltpu.sync_copy(buf0.at[pl.ds(0, _NB)], out_hbm.at[wid])


_sc_compiler_params = pltpu.CompilerParams()
if "needs_layout_passes" in pltpu.CompilerParams.__dataclass_fields__:
    _sc_compiler_params = dataclasses.replace(
        _sc_compiler_params, needs_layout_passes=False)

_sc_hist_call = functools.partial(
    pl.kernel,
    compiler_params=_sc_compiler_params,
    out_type=jax.ShapeDtypeStruct((_NW, _NB), jnp.float32),
    mesh=plsc.VectorSubcoreMesh(core_axis_name="c", subcore_axis_name="s"),
    scratch_types=[
        pltpu.VMEM((2 * _L,), jnp.int32),
        pltpu.VMEM((_CHUNK,), jnp.float32),
        pltpu.VMEM((_CHUNK,), jnp.float32),
        pltpu.VMEM((_NB * _L,), jnp.float32),
        pltpu.SemaphoreType.DMA,
        pltpu.SemaphoreType.DMA,
    ],
)(_sc_hist_body)


def _sc_histogram(w_flat, lo, shift):
    lo_arr = jnp.concatenate([
        jnp.full((_L,), lo, jnp.int32), jnp.full((_L,), shift, jnp.int32)])
    out = _sc_hist_call(lo_arr, w_flat)
    return out.sum(axis=0)


def _select_threshold(w_flat):
    shifts = jnp.array([19, 8, 0], jnp.int32)

    def body(p, carry):
        lo, rank = carry
        shift = shifts[p]
        counts = _sc_histogram(w_flat, lo, shift)
        cum = jnp.cumsum(counts)
        need = (_K_INDEX + 1 - rank).astype(jnp.float32)
        j = jnp.argmax(cum >= need).astype(jnp.int32)
        below = jnp.where(j > 0, cum[jnp.maximum(j - 1, 0)], 0.0)
        lo = lo + lax.shift_left(j, shift)
        rank = rank + below.astype(jnp.int32)
        return lo, rank

    lo, _ = jax.lax.fori_loop(0, 3, body, (jnp.int32(0), jnp.int32(0)))
    return jax.lax.bitcast_convert_type(lo, jnp.float32)


def _mm_kernel(thr_ref, x_ref, w_ref, b_ref, out_ref):
    thr = thr_ref[0]
    w = w_ref[...]
    wm = jnp.where(jnp.abs(w) >= thr, w, 0.0).astype(jnp.bfloat16)
    acc = jax.lax.dot_general(x_ref[...], wm, (((1,), (1,)), ((), ())),
                              preferred_element_type=jnp.float32)
    out_ref[...] = acc + b_ref[...]


def _masked_matmul(x_bf, weight, bias2d, thr, block_o=512):
    n_tok, d_in = x_bf.shape
    d_out = weight.shape[0]
    return pl.pallas_call(
        _mm_kernel,
        grid=(d_out // block_o,),
        in_specs=[
            pl.BlockSpec(memory_space=pltpu.SMEM),
            pl.BlockSpec((n_tok, d_in), lambda i: (0, 0)),
            pl.BlockSpec((block_o, d_in), lambda i: (i, 0)),
            pl.BlockSpec((1, block_o), lambda i: (0, i)),
        ],
        out_specs=pl.BlockSpec((n_tok, block_o), lambda i: (0, i)),
        out_shape=jax.ShapeDtypeStruct((n_tok, d_out), jnp.float32),
    )(jnp.reshape(thr, (1,)), x_bf, weight, bias2d)


@jax.jit
def kernel(inputs, weight, bias):
    thr = _select_threshold(jnp.reshape(weight, (-1,)))
    x_bf = inputs.astype(jnp.bfloat16)
    return _masked_matmul(x_bf, weight, jnp.reshape(bias, (1, -1)), thr)


# speedup vs baseline: 2.8922x; 2.8922x over previous
import dataclasses
import functools

import jax
import jax.numpy as jnp
from jax import lax
from jax.experimental import pallas as pl
from jax.experimental.pallas import tpu as pltpu
from jax.experimental.pallas import tpu_sc as plsc

_K_INDEX = 15938354

_NB = 2048
_L = 16
_NW = 32
_ELEMS = 4096 * 4096
_ROWS = 4096
_COLS = 4096
_ROWS_W = _ROWS // _NW
_CROWS = 8
_CHUNK = _CROWS * _COLS
_NCHUNK = _ROWS_W // _CROWS


def _sc_hist_body(lo_hbm, w_hbm, out_hbm, lo_v, buf0, buf1, hist, sem0, sem1):
    wid = lax.axis_index("s") * 2 + lax.axis_index("c")
    rbase = wid * _ROWS_W

    pltpu.sync_copy(lo_hbm, lo_v)
    lovec = lo_v[pl.ds(0, _L)]
    shvec = lo_v[pl.ds(_L, _L)]
    lane = lax.iota(jnp.int32, _L)
    four = jnp.full((_L,), 4, jnp.int32)
    ones = jnp.full((_L,), 1.0, jnp.float32)
    nbm1 = jnp.full((_L,), _NB - 1, jnp.int32)
    zero = jnp.zeros((_L,), jnp.int32)

    @plsc.parallel_loop(0, (_NB * _L) // _L, unroll=8)
    def _z(i):
        hist[pl.ds(i * _L, _L)] = jnp.zeros((_L,), jnp.float32)

    bufs = (buf0, buf1)
    sems = (sem0, sem1)
    copies = [None, None]
    copies[0] = pltpu.async_copy(w_hbm.at[pl.ds(rbase, _CROWS)], buf0, sem0)

    def _process(buf):
        for r in range(_CROWS):
            @plsc.parallel_loop(0, _COLS // _L, unroll=8)
            def _b(i):
                v = buf[r, pl.ds(i * _L, _L)]
                bits = plsc.bitcast(jnp.abs(v), jnp.int32)
                rel = lax.shift_right_arithmetic(bits - lovec, shvec)
                rel = jnp.minimum(jnp.maximum(rel, zero), nbm1)
                addr = lax.shift_left(rel, four) + lane
                plsc.addupdate_scatter(hist, [addr], ones)

    for c in range(_NCHUNK):
        if c + 1 < _NCHUNK:
            copies[(c + 1) % 2] = pltpu.async_copy(
                w_hbm.at[pl.ds(rbase + (c + 1) * _CROWS, _CROWS)],
                bufs[(c + 1) % 2], sems[(c + 1) % 2])
        copies[c % 2].wait()
        _process(bufs[c % 2])

    @plsc.parallel_loop(0, _NB // _L, unroll=2)
    def _r(i):
        kbase = i * _L
        acc = jnp.zeros((_L,), jnp.float32)
        gidx = lax.shift_left(kbase + lane, four)
        for l in range(_L):
            acc = acc + plsc.load_gather(hist, [gidx + l])
        buf0[0, pl.ds(kbase, _L)] = acc

    pltpu.sync_copy(buf0.at[0, pl.ds(0, _NB)], out_hbm.at[wid])


_sc_compiler_params = pltpu.CompilerParams()
if "needs_layout_passes" in pltpu.CompilerParams.__dataclass_fields__:
    _sc_compiler_params = dataclasses.replace(
        _sc_compiler_params, needs_layout_passes=False)

_sc_hist_call = functools.partial(
    pl.kernel,
    compiler_params=_sc_compiler_params,
    out_type=jax.ShapeDtypeStruct((_NW, _NB), jnp.float32),
    mesh=plsc.VectorSubcoreMesh(core_axis_name="c", subcore_axis_name="s"),
    scratch_types=[
        pltpu.VMEM((2 * _L,), jnp.int32),
        pltpu.VMEM((_CROWS, _COLS), jnp.float32),
        pltpu.VMEM((_CROWS, _COLS), jnp.float32),
        pltpu.VMEM((_NB * _L,), jnp.float32),
        pltpu.SemaphoreType.DMA,
        pltpu.SemaphoreType.DMA,
    ],
)(_sc_hist_body)


def _sc_histogram(weight, lo, shift):
    lo_arr = jnp.concatenate([
        jnp.full((_L,), lo, jnp.int32), jnp.full((_L,), shift, jnp.int32)])
    out = _sc_hist_call(lo_arr, weight)
    return out.sum(axis=0)


def _select_threshold(weight):
    shifts = jnp.array([19, 8, 0], jnp.int32)

    def body(p, lo):
        shift = shifts[p]
        counts = _sc_histogram(weight, lo, shift)
        cum = jnp.cumsum(counts)
        j = jnp.argmax(cum >= jnp.float32(_K_INDEX + 1)).astype(jnp.int32)
        return lo + lax.shift_left(j, shift)

    lo = jax.lax.fori_loop(0, 3, body, jnp.int32(0))
    return jax.lax.bitcast_convert_type(lo, jnp.float32)


def _mm_kernel(thr_ref, x_ref, w_ref, b_ref, out_ref):
    thr = thr_ref[0]
    w = w_ref[...]
    wm = jnp.where(jnp.abs(w) >= thr, w, 0.0).astype(jnp.bfloat16)
    acc = jax.lax.dot_general(x_ref[...], wm, (((1,), (1,)), ((), ())),
                              preferred_element_type=jnp.float32)
    out_ref[...] = acc + b_ref[...]


def _masked_matmul(x_bf, weight, bias2d, thr, block_o=512):
    n_tok, d_in = x_bf.shape
    d_out = weight.shape[0]
    return pl.pallas_call(
        _mm_kernel,
        grid=(d_out // block_o,),
        in_specs=[
            pl.BlockSpec(memory_space=pltpu.SMEM),
            pl.BlockSpec((n_tok, d_in), lambda i: (0, 0)),
            pl.BlockSpec((block_o, d_in), lambda i: (i, 0)),
            pl.BlockSpec((1, block_o), lambda i: (0, i)),
        ],
        out_specs=pl.BlockSpec((n_tok, block_o), lambda i: (0, i)),
        out_shape=jax.ShapeDtypeStruct((n_tok, d_out), jnp.float32),
    )(jnp.reshape(thr, (1,)), x_bf, weight, bias2d)


@jax.jit
def kernel(inputs, weight, bias):
    thr = _select_threshold(weight)
    x_bf = inputs.astype(jnp.bfloat16)
    return _masked_matmul(x_bf, weight, jnp.reshape(bias, (1, -1)), thr)
